# baseline (speedup 1.0000x reference)
import dataclasses
import functools

import jax
import jax.numpy as jnp
from jax import lax
from jax.experimental import pallas as pl
from jax.experimental.pallas import tpu as pltpu
from jax.experimental.pallas import tpu_sc as plsc

_MAX_ITERS = 10
_THRESHOLD = 1e-06

_NC = 2
_NS = 16
_NW = _NC * _NS
_LANES = 16
_CHUNK = 512
_NBUF = 2


def _fast_log(q):
    bits = plsc.bitcast(q, jnp.int32)
    u = bits + (0x3F800000 - 0x3F3504F3)
    ex = (u >> 23) - 127
    m = plsc.bitcast((u & 0x007FFFFF) + 0x3F3504F3, jnp.float32)
    z = (m - 1.0) / (m + 1.0)
    z2 = z * z
    p = 2.0 + z2 * (0.66666667 + z2 * (
        0.4 + z2 * (0.2857143 + z2 * 0.22222222)))
    return ex.astype(jnp.float32) * 0.6931472 + z * p


def _fast_exp(x):
    kf = (x * 1.4426950408889634 + 16384.5).astype(jnp.int32) - 16384
    k = kf.astype(jnp.float32)
    f = (x - k * 0.693359375) + k * 2.12194440e-4
    p = 1.0 + f * (1.0 + f * (0.5 + f * (0.16666667 + f * (
        0.041666668 + f * (0.008333452 + f * 0.0013887406)))))
    scale = plsc.bitcast((kf + 127) << 23, jnp.float32)
    return p * scale


def _edge_update(r, lother, av, bv, cv, dv):
    x = jnp.minimum(jnp.maximum(r - lother, -25.0), 25.0)
    tau = _fast_exp(x)
    return _fast_log((av * tau + bv) / (cv * tau + dv))


def _make_edge_step(epad, npad, cpt):
    mesh = plsc.VectorSubcoreMesh(core_axis_name="c", subcore_axis_name="s")
    fdt = jnp.float32
    idt = jnp.int32
    cp = pltpu.CompilerParams()
    if "needs_layout_passes" in pltpu.CompilerParams.__dataclass_fields__:
        cp = dataclasses.replace(cp, needs_layout_passes=False)

    @functools.partial(
        pl.kernel,
        out_type=(
            jax.ShapeDtypeStruct((epad,), fdt),
            jax.ShapeDtypeStruct((epad,), fdt),
            jax.ShapeDtypeStruct((_NC, npad), fdt),
        ),
        mesh=mesh,
        compiler_params=cp,
        scratch_types=[
            pltpu.VMEM((npad,), fdt),
            pltpu.VMEM((64,), fdt),
        ] + [
            pltpu.VMEM((_CHUNK,), idt) for _ in range(4)
        ] + [
            pltpu.VMEM((_CHUNK,), t)
            for _ in range(_NBUF)
            for t in (idt, fdt, fdt, fdt, fdt)
        ] + [
            pltpu.VMEM_SHARED((npad,), fdt),
            pltpu.SemaphoreType.DMA((_NBUF,)),
            pltpu.SemaphoreType.DMA((_NBUF,)),
            pltpu.SemaphoreType.DMA((_NBUF,)),
        ],
    )
    def edge_step(lrho_hbm, src_hbm, dst_hbm, lmu_hbm, lnu_hbm, pots_hbm,
                  zeros_hbm, lmu_out, lnu_out, accs_out,
                  tab_v, pots_v, *bufs_and_rest):
        srcs = bufs_and_rest[:4]
        rest = bufs_and_rest[4:]
        bufs = [rest[5 * b:5 * b + 5] for b in range(_NBUF)]
        acc_sh, sem_in, sem_out, sem_sc = rest[5 * _NBUF:]
        cid = lax.axis_index("c")
        sid = lax.axis_index("s")
        wid = cid * _NS + sid
        tile_base = wid * cpt

        pltpu.sync_copy(lrho_hbm, tab_v)
        pltpu.sync_copy(pots_hbm, pots_v)

        @pl.when(sid == 0)
        def _():
            pltpu.sync_copy(zeros_hbm, acc_sh)

        plsc.subcore_barrier()

        av = pots_v[pl.ds(0, _LANES)]
        bv = pots_v[pl.ds(16, _LANES)]
        cv = pots_v[pl.ds(32, _LANES)]
        dv = pots_v[pl.ds(48, _LANES)]

        def issue_in(cg, db, sb):
            dst_c, lmu_c, lnu_c, _, _ = bufs[db]
            base = (tile_base + cg) * _CHUNK
            sl = pl.ds(base, _CHUNK)
            pltpu.async_copy(src_hbm.at[sl], srcs[sb], sem_in.at[db])
            pltpu.async_copy(dst_hbm.at[sl], dst_c, sem_in.at[db])
            pltpu.async_copy(lmu_hbm.at[sl], lmu_c, sem_in.at[db])
            pltpu.async_copy(lnu_hbm.at[sl], lnu_c, sem_in.at[db])

        def wait_in(db, sb):
            dst_c, lmu_c, lnu_c, _, _ = bufs[db]
            sl = pl.ds(0, _CHUNK)
            pltpu.make_async_copy(src_hbm.at[sl], srcs[sb], sem_in.at[db]).wait()
            pltpu.make_async_copy(dst_hbm.at[sl], dst_c, sem_in.at[db]).wait()
            pltpu.make_async_copy(lmu_hbm.at[sl], lmu_c, sem_in.at[db]).wait()
            pltpu.make_async_copy(lnu_hbm.at[sl], lnu_c, sem_in.at[db]).wait()

        def wait_out(db, sb):
            _, _, _, lmu_o, lnu_o = bufs[db]
            sl = pl.ds(0, _CHUNK)
            pltpu.make_async_copy(lmu_o, lmu_out.at[sl], sem_out.at[db]).wait()
            pltpu.make_async_copy(lnu_o, lnu_out.at[sl], sem_out.at[db]).wait()
            pltpu.make_async_copy(
                lnu_o, acc_sh.at[srcs[sb]], sem_sc.at[db]).wait()

        for b in range(_NBUF):
            issue_in(b, b, b)

        @pl.loop(0, cpt, step=4)
        def _(g):
            for b in range(4):
                cg = g + b
                db = b % _NBUF
                src_c = srcs[b]
                dst_c, lmu_c, lnu_c, lmu_o, lnu_o = bufs[db]
                wait_in(db, b)

                @pl.when(cg >= _NBUF)
                def _():
                    wait_out(db, (b + 2) % 4)

                @plsc.parallel_loop(0, _CHUNK, step=_LANES, unroll=4)
                def _(k):
                    s16 = src_c[pl.ds(k, _LANES)]
                    d16 = dst_c[pl.ds(k, _LANES)]
                    lmu16 = lmu_c[pl.ds(k, _LANES)]
                    lnu16 = lnu_c[pl.ds(k, _LANES)]
                    rs = plsc.load_gather(tab_v, [s16])
                    rd = plsc.load_gather(tab_v, [d16])
                    lmu_o[pl.ds(k, _LANES)] = _edge_update(
                        rs, lnu16, av, bv, cv, dv)
                    lnu_o[pl.ds(k, _LANES)] = _edge_update(
                        rd, lmu16, av, bv, cv, dv)

                base = (tile_base + cg) * _CHUNK
                sl = pl.ds(base, _CHUNK)
                pltpu.async_copy(lmu_o, lmu_out.at[sl], sem_out.at[db])
                pltpu.async_copy(lnu_o, lnu_out.at[sl], sem_out.at[db])
                pltpu.async_copy(lnu_o, acc_sh.at[src_c], sem_sc.at[db],
                                 add=True)

                @pl.when(cg + _NBUF < cpt)
                def _():
                    issue_in(cg + _NBUF, db, (b + 2) % 4)

        wait_out(0, 2)
        wait_out(1, 3)

        plsc.subcore_barrier()

        @pl.when(sid == 0)
        def _():
            pltpu.sync_copy(acc_sh, accs_out.at[cid])

    return edge_step


def kernel(priors, potential, src_nodes, dst_nodes, rev_edges):
    n = priors.shape[0]
    e = src_nodes.shape[0]
    eblk = _CHUNK * _NW * 4
    epad = ((e + eblk - 1) // eblk) * eblk
    cpt = epad // (_CHUNK * _NW)
    npad = ((n + 1 + 127) // 128) * 128

    fdt = priors.dtype
    src_p = jnp.full((epad,), n, jnp.int32).at[:e].set(src_nodes.astype(jnp.int32))
    dst_p = jnp.full((epad,), n, jnp.int32).at[:e].set(dst_nodes.astype(jnp.int32))
    lpr = jnp.log(priors[:, 0]) - jnp.log(priors[:, 1])
    pots = jnp.concatenate([
        jnp.full((16,), potential[0, 0], fdt),
        jnp.full((16,), potential[1, 0], fdt),
        jnp.full((16,), potential[0, 1], fdt),
        jnp.full((16,), potential[1, 1], fdt),
    ])
    zeros_n = jnp.zeros((npad,), fdt)
    lmu0 = jnp.zeros((epad,), fdt)
    lrho0 = lpr

    edge_step = _make_edge_step(epad, npad, cpt)

    def cond_fn(state):
        i, _, _, _, diff = state
        return jnp.logical_and(i < _MAX_ITERS, diff >= _THRESHOLD)

    def body_fn(state):
        i, lrho, lmu, lnu, _ = state
        tab = jnp.zeros((npad,), fdt).at[:n].set(lrho)
        lmu_n, lnu_n, accs = edge_step(tab, src_p, dst_p, lmu, lnu, pots,
                                       zeros_n)
        lrho_n = lpr + accs[0, :n] + accs[1, :n]
        diff = jnp.max(jnp.abs(jax.nn.sigmoid(lrho_n) - jax.nn.sigmoid(lrho)))
        return (i + 1, lrho_n, lmu_n, lnu_n, diff)

    init = (0, lrho0, lmu0, lmu0, jnp.asarray(jnp.inf, fdt))
    _, lrho, _, _, _ = lax.while_loop(cond_fn, body_fn, init)
    return jnp.stack([jax.nn.sigmoid(lrho), jax.nn.sigmoid(-lrho)], axis=1)


# speedup vs baseline: 1.3677x; 1.3677x over previous
import dataclasses
import functools

import jax
import jax.numpy as jnp
from jax import lax
from jax.experimental import pallas as pl
from jax.experimental.pallas import tpu as pltpu
from jax.experimental.pallas import tpu_sc as plsc

_MAX_ITERS = 10
_THRESHOLD = 1e-06

_NC = 2
_NS = 16
_NW = _NC * _NS
_LANES = 16
_CHUNK = 512
_NBUF = 2


def _fast_log(q):
    bits = plsc.bitcast(q, jnp.int32)
    u = bits + (0x3F800000 - 0x3F3504F3)
    ex = (u >> 23) - 127
    m = plsc.bitcast((u & 0x007FFFFF) + 0x3F3504F3, jnp.float32)
    z = (m - 1.0) / (m + 1.0)
    z2 = z * z
    p = 2.0 + z2 * (0.66666667 + z2 * 0.4)
    return ex.astype(jnp.float32) * 0.6931472 + z * p


def _edge_update(r, lother, av, bv, cv, dv):
    x = jnp.minimum(jnp.maximum(r - lother, -25.0), 25.0)
    tau = jnp.exp(x)
    return _fast_log((av * tau + bv) / (cv * tau + dv))


def _make_edge_step(epad, npad, cpt):
    mesh = plsc.VectorSubcoreMesh(core_axis_name="c", subcore_axis_name="s")
    fdt = jnp.float32
    idt = jnp.int32
    cp = pltpu.CompilerParams()
    if "needs_layout_passes" in pltpu.CompilerParams.__dataclass_fields__:
        cp = dataclasses.replace(cp, needs_layout_passes=False)

    @functools.partial(
        pl.kernel,
        out_type=(
            jax.ShapeDtypeStruct((epad,), fdt),
            jax.ShapeDtypeStruct((epad,), fdt),
            jax.ShapeDtypeStruct((_NC, npad), fdt),
        ),
        mesh=mesh,
        compiler_params=cp,
        scratch_types=[
            pltpu.VMEM((npad,), fdt),
            pltpu.VMEM((64,), fdt),
        ] + [
            pltpu.VMEM((_CHUNK,), idt) for _ in range(4)
        ] + [
            pltpu.VMEM((_CHUNK,), t)
            for _ in range(_NBUF)
            for t in (idt, fdt, fdt, fdt, fdt)
        ] + [
            pltpu.VMEM_SHARED((npad,), fdt),
            pltpu.SemaphoreType.DMA((_NBUF,)),
            pltpu.SemaphoreType.DMA((_NBUF,)),
            pltpu.SemaphoreType.DMA((_NBUF,)),
        ],
    )
    def edge_step(lrho_hbm, src_hbm, dst_hbm, lmu_hbm, lnu_hbm, pots_hbm,
                  zeros_hbm, lmu_out, lnu_out, accs_out,
                  tab_v, pots_v, *bufs_and_rest):
        srcs = bufs_and_rest[:4]
        rest = bufs_and_rest[4:]
        bufs = [rest[5 * b:5 * b + 5] for b in range(_NBUF)]
        acc_sh, sem_in, sem_out, sem_sc = rest[5 * _NBUF:]
        cid = lax.axis_index("c")
        sid = lax.axis_index("s")
        wid = cid * _NS + sid
        tile_base = wid * cpt

        pltpu.sync_copy(lrho_hbm, tab_v)
        pltpu.sync_copy(pots_hbm, pots_v)

        @pl.when(sid == 0)
        def _():
            pltpu.sync_copy(zeros_hbm, acc_sh)

        plsc.subcore_barrier()

        av = pots_v[pl.ds(0, _LANES)]
        bv = pots_v[pl.ds(16, _LANES)]
        cv = pots_v[pl.ds(32, _LANES)]
        dv = pots_v[pl.ds(48, _LANES)]

        def issue_in(cg, db, sb):
            dst_c, lmu_c, lnu_c, _, _ = bufs[db]
            base = (tile_base + cg) * _CHUNK
            sl = pl.ds(base, _CHUNK)
            pltpu.async_copy(src_hbm.at[sl], srcs[sb], sem_in.at[db])
            pltpu.async_copy(dst_hbm.at[sl], dst_c, sem_in.at[db])
            pltpu.async_copy(lmu_hbm.at[sl], lmu_c, sem_in.at[db])
            pltpu.async_copy(lnu_hbm.at[sl], lnu_c, sem_in.at[db])

        def wait_in(db, sb):
            dst_c, lmu_c, lnu_c, _, _ = bufs[db]
            sl = pl.ds(0, _CHUNK)
            pltpu.make_async_copy(src_hbm.at[sl], srcs[sb], sem_in.at[db]).wait()
            pltpu.make_async_copy(dst_hbm.at[sl], dst_c, sem_in.at[db]).wait()
            pltpu.make_async_copy(lmu_hbm.at[sl], lmu_c, sem_in.at[db]).wait()
            pltpu.make_async_copy(lnu_hbm.at[sl], lnu_c, sem_in.at[db]).wait()

        def wait_out(db, sb):
            _, _, _, lmu_o, lnu_o = bufs[db]
            sl = pl.ds(0, _CHUNK)
            pltpu.make_async_copy(lmu_o, lmu_out.at[sl], sem_out.at[db]).wait()
            pltpu.make_async_copy(lnu_o, lnu_out.at[sl], sem_out.at[db]).wait()
            pltpu.make_async_copy(
                lnu_o, acc_sh.at[srcs[sb]], sem_sc.at[db]).wait()

        for b in range(_NBUF):
            issue_in(b, b, b)

        @pl.loop(0, cpt, step=4)
        def _(g):
            for b in range(4):
                cg = g + b
                db = b % _NBUF
                src_c = srcs[b]
                dst_c, lmu_c, lnu_c, lmu_o, lnu_o = bufs[db]
                wait_in(db, b)

                @pl.when(cg >= _NBUF)
                def _():
                    wait_out(db, (b + 2) % 4)

                @plsc.parallel_loop(0, _CHUNK, step=_LANES, unroll=4)
                def _(k):
                    s16 = src_c[pl.ds(k, _LANES)]
                    d16 = dst_c[pl.ds(k, _LANES)]
                    lmu16 = lmu_c[pl.ds(k, _LANES)]
                    lnu16 = lnu_c[pl.ds(k, _LANES)]
                    rs = plsc.load_gather(tab_v, [s16])
                    rd = plsc.load_gather(tab_v, [d16])
                    lmu_o[pl.ds(k, _LANES)] = _edge_update(
                        rs, lnu16, av, bv, cv, dv)
                    lnu_o[pl.ds(k, _LANES)] = _edge_update(
                        rd, lmu16, av, bv, cv, dv)

                base = (tile_base + cg) * _CHUNK
                sl = pl.ds(base, _CHUNK)
                pltpu.async_copy(lmu_o, lmu_out.at[sl], sem_out.at[db])
                pltpu.async_copy(lnu_o, lnu_out.at[sl], sem_out.at[db])
                pltpu.async_copy(lnu_o, acc_sh.at[src_c], sem_sc.at[db],
                                 add=True)

                @pl.when(cg + _NBUF < cpt)
                def _():
                    issue_in(cg + _NBUF, db, (b + 2) % 4)

        wait_out(0, 2)
        wait_out(1, 3)

        plsc.subcore_barrier()

        @pl.when(sid == 0)
        def _():
            pltpu.sync_copy(acc_sh, accs_out.at[cid])

    return edge_step


def kernel(priors, potential, src_nodes, dst_nodes, rev_edges):
    n = priors.shape[0]
    e = src_nodes.shape[0]
    eblk = _CHUNK * _NW * 4
    epad = ((e + eblk - 1) // eblk) * eblk
    cpt = epad // (_CHUNK * _NW)
    npad = ((n + 1 + 127) // 128) * 128

    fdt = priors.dtype
    src_p = jnp.full((epad,), n, jnp.int32).at[:e].set(src_nodes.astype(jnp.int32))
    dst_p = jnp.full((epad,), n, jnp.int32).at[:e].set(dst_nodes.astype(jnp.int32))
    lpr = jnp.log(priors[:, 0]) - jnp.log(priors[:, 1])
    pots = jnp.concatenate([
        jnp.full((16,), potential[0, 0], fdt),
        jnp.full((16,), potential[1, 0], fdt),
        jnp.full((16,), potential[0, 1], fdt),
        jnp.full((16,), potential[1, 1], fdt),
    ])
    zeros_n = jnp.zeros((npad,), fdt)
    lmu0 = jnp.zeros((epad,), fdt)
    lrho0 = lpr

    edge_step = _make_edge_step(epad, npad, cpt)

    def cond_fn(state):
        i, _, _, _, diff = state
        return jnp.logical_and(i < _MAX_ITERS, diff >= _THRESHOLD)

    def body_fn(state):
        i, lrho, lmu, lnu, _ = state
        tab = jnp.zeros((npad,), fdt).at[:n].set(lrho)
        lmu_n, lnu_n, accs = edge_step(tab, src_p, dst_p, lmu, lnu, pots,
                                       zeros_n)
        lrho_n = lpr + accs[0, :n] + accs[1, :n]
        diff = jnp.max(jnp.abs(jax.nn.sigmoid(lrho_n) - jax.nn.sigmoid(lrho)))
        return (i + 1, lrho_n, lmu_n, lnu_n, diff)

    init = (0, lrho0, lmu0, lmu0, jnp.asarray(jnp.inf, fdt))
    _, lrho, _, _, _ = lax.while_loop(cond_fn, body_fn, init)
    return jnp.stack([jax.nn.sigmoid(lrho), jax.nn.sigmoid(-lrho)], axis=1)
